# Initial kernel scaffold; baseline (speedup 1.0000x reference)
import jax
import jax.numpy as jnp
from jax.experimental import pallas as pl


def kernel(x, edge_index, W_self1, W_neigh1, b1, W_self2, W_neigh2, b2):
    raise NotImplementedError("write your pallas kernel here")


# speedup vs baseline: 6.2832x; 6.2832x over previous
import functools

import jax
import jax.numpy as jnp
from jax import lax
from jax.experimental import pallas as pl
from jax.experimental.pallas import tpu as pltpu
from jax.experimental.pallas import tpu_sc as plsc

N = 10000
E = 320000
D = 128
NCLS = 40
CP = 48
NC = 2
NS = 16
NW = NC * NS
EPW = E // NW
CHUNK = 128
NCH = (EPW + CHUNK - 1) // CHUNK
EPW_PAD = NCH * CHUNK
DH = D // NC
EPS = E // NS
NCH1 = (EPS + CHUNK - 1) // CHUNK
EPS_PAD = NCH1 * CHUNK
NPAD = N + 112
STRIPE = NPAD // NS
CW = 16

_MESH = dict(core_axis_name="c", subcore_axis_name="s")


def _sc_agg1(xstk, src2, dst1, zh, zc, onesc):
    @functools.partial(
        pl.kernel,
        mesh=plsc.VectorSubcoreMesh(**_MESH),
        compiler_params=pltpu.CompilerParams(use_tc_tiling_on_sc=False),
        out_type=(
            jax.ShapeDtypeStruct((NC, NPAD, DH), jnp.float32),
            jax.ShapeDtypeStruct((NPAD, CW), jnp.float32),
        ),
        scratch_types=[
            pltpu.VMEM((NCH1, CHUNK), jnp.int32),
            pltpu.VMEM((NCH1, CHUNK), jnp.int32),
            pltpu.VMEM((CHUNK, DH), jnp.float32),
            pltpu.VMEM((CHUNK, CW), jnp.float32),
            pltpu.VMEM_SHARED((NPAD, DH), jnp.float32),
            pltpu.VMEM_SHARED((NPAD, CW), jnp.float32),
            pltpu.SemaphoreType.DMA,
        ],
    )
    def k(x_hbm, src_hbm, dst_hbm, zh_hbm, zc_hbm, ones_hbm,
          sums_hbm, cnts_hbm,
          srcv, dstv, rows, onesv, acc, cacc, sem):
        cid = lax.axis_index("c")
        sid = lax.axis_index("s")
        base = sid * STRIPE
        pltpu.sync_copy(zh_hbm, acc.at[pl.ds(base, STRIPE)])
        pltpu.sync_copy(zc_hbm, cacc.at[pl.ds(base, STRIPE)])
        pltpu.sync_copy(src_hbm.at[cid, sid], srcv)
        pltpu.sync_copy(dst_hbm.at[sid], dstv)
        pltpu.sync_copy(ones_hbm, onesv)
        plsc.subcore_barrier()

        def body(j, carry):
            pltpu.async_copy(x_hbm.at[srcv.at[j]], rows, sem).wait()
            pltpu.sync_copy(rows, acc.at[dstv.at[j]], add=True)
            return carry

        lax.fori_loop(0, NCH1, body, 0)

        @pl.when(cid == 0)
        def _():
            def cbody(j, carry):
                pltpu.sync_copy(onesv, cacc.at[dstv.at[j]], add=True)
                return carry
            lax.fori_loop(0, NCH1, cbody, 0)

        plsc.subcore_barrier()
        pltpu.sync_copy(acc.at[pl.ds(base, STRIPE)],
                        sums_hbm.at[cid, pl.ds(base, STRIPE)])

        @pl.when(cid == 0)
        def _():
            pltpu.sync_copy(cacc.at[pl.ds(base, STRIPE)],
                            cnts_hbm.at[pl.ds(base, STRIPE)])

    return k(xstk, src2, dst1, zh, zc, onesc)


def _sc_agg2(p2, srcw, dstw, zcp):
    @functools.partial(
        pl.kernel,
        mesh=plsc.VectorSubcoreMesh(**_MESH),
        compiler_params=pltpu.CompilerParams(use_tc_tiling_on_sc=False),
        out_type=jax.ShapeDtypeStruct((NC, NPAD, CP), jnp.float32),
        scratch_types=[
            pltpu.VMEM((NCH, CHUNK), jnp.int32),
            pltpu.VMEM((NCH, CHUNK), jnp.int32),
            pltpu.VMEM((CHUNK, CP), jnp.float32),
            pltpu.VMEM_SHARED((NPAD, CP), jnp.float32),
            pltpu.SemaphoreType.DMA,
        ],
    )
    def k(p2_hbm, src_hbm, dst_hbm, zcp_hbm,
          sums_hbm,
          srcv, dstv, rows, acc, sem):
        cid = lax.axis_index("c")
        sid = lax.axis_index("s")
        wid = sid * NC + cid
        base = sid * STRIPE
        pltpu.sync_copy(zcp_hbm, acc.at[pl.ds(base, STRIPE)])
        pltpu.sync_copy(src_hbm.at[wid], srcv)
        pltpu.sync_copy(dst_hbm.at[wid], dstv)
        plsc.subcore_barrier()

        def body(j, carry):
            pltpu.async_copy(p2_hbm.at[srcv.at[j]], rows, sem).wait()
            pltpu.sync_copy(rows, acc.at[dstv.at[j]], add=True)
            return carry

        lax.fori_loop(0, NCH, body, 0)
        plsc.subcore_barrier()
        pltpu.sync_copy(acc.at[pl.ds(base, STRIPE)],
                        sums_hbm.at[cid, pl.ds(base, STRIPE)])

    return k(p2, srcw, dstw, zcp)


_R = 400


def _tc_mid(x, sums1, cnts, Ws1, Wn1, b1r, Ws2p, Wn2p, b2p):
    def body(x_ref, s_ref, c_ref, ws1_ref, wn1_ref, b1_ref,
             ws2_ref, wn2_ref, b2_ref, p2_ref, s2_ref):
        s = jnp.concatenate([s_ref[0], s_ref[1]], axis=1)
        c = c_ref[:, 0:1]
        mean = s / jnp.maximum(c, 1.0)
        h = jnp.dot(x_ref[...], ws1_ref[...], preferred_element_type=jnp.float32)
        h = h + jnp.dot(mean, wn1_ref[...], preferred_element_type=jnp.float32)
        h = jnp.maximum(h + b1_ref[...], 0.0)
        p2_ref[...] = jnp.dot(h, wn2_ref[...], preferred_element_type=jnp.float32)
        s2_ref[...] = jnp.dot(h, ws2_ref[...],
                              preferred_element_type=jnp.float32) + b2_ref[...]

    return pl.pallas_call(
        body,
        grid=(N // _R,),
        in_specs=[
            pl.BlockSpec((_R, D), lambda i: (i, 0)),
            pl.BlockSpec((NC, _R, DH), lambda i: (0, i, 0)),
            pl.BlockSpec((_R, CW), lambda i: (i, 0)),
            pl.BlockSpec((D, D), lambda i: (0, 0)),
            pl.BlockSpec((D, D), lambda i: (0, 0)),
            pl.BlockSpec((1, D), lambda i: (0, 0)),
            pl.BlockSpec((D, CP), lambda i: (0, 0)),
            pl.BlockSpec((D, CP), lambda i: (0, 0)),
            pl.BlockSpec((1, CP), lambda i: (0, 0)),
        ],
        out_specs=[
            pl.BlockSpec((_R, CP), lambda i: (i, 0)),
            pl.BlockSpec((_R, CP), lambda i: (i, 0)),
        ],
        out_shape=[
            jax.ShapeDtypeStruct((N, CP), jnp.float32),
            jax.ShapeDtypeStruct((N, CP), jnp.float32),
        ],
    )(x, sums1, cnts, Ws1, Wn1, b1r, Ws2p, Wn2p, b2p)


def _tc_out(s2, sums2, cnts):
    def body(s2_ref, s_ref, c_ref, o_ref):
        s = s_ref[0] + s_ref[1]
        c = c_ref[:, 0:1]
        o_ref[...] = (s2_ref[...] + s / jnp.maximum(c, 1.0))[:, :NCLS]

    return pl.pallas_call(
        body,
        grid=(N // _R,),
        in_specs=[
            pl.BlockSpec((_R, CP), lambda i: (i, 0)),
            pl.BlockSpec((NC, _R, CP), lambda i: (0, i, 0)),
            pl.BlockSpec((_R, CW), lambda i: (i, 0)),
        ],
        out_specs=pl.BlockSpec((_R, NCLS), lambda i: (i, 0)),
        out_shape=jax.ShapeDtypeStruct((N, NCLS), jnp.float32),
    )(s2, sums2, cnts)


def kernel(x, edge_index, W_self1, W_neigh1, b1, W_self2, W_neigh2, b2):
    src = edge_index[0]
    dst = edge_index[1]
    xstk = jnp.concatenate([x[:, :DH], x[:, DH:]], axis=0)
    srcp = jnp.pad(src.reshape(NS, EPS),
                   ((0, 0), (0, EPS_PAD - EPS))).reshape(NS, NCH1, CHUNK)
    src2 = jnp.stack([srcp, srcp + N])
    dst1 = jnp.pad(dst.reshape(NS, EPS), ((0, 0), (0, EPS_PAD - EPS)),
                   constant_values=N).reshape(NS, NCH1, CHUNK)
    srcw = jnp.pad(src.reshape(NW, EPW),
                   ((0, 0), (0, EPW_PAD - EPW))).reshape(NW, NCH, CHUNK)
    dstw = jnp.pad(dst.reshape(NW, EPW), ((0, 0), (0, EPW_PAD - EPW)),
                   constant_values=N).reshape(NW, NCH, CHUNK)
    zh = jnp.zeros((STRIPE, DH), jnp.float32)
    zc = jnp.zeros((STRIPE, CW), jnp.float32)
    zcp = jnp.zeros((STRIPE, CP), jnp.float32)
    onesc = jnp.zeros((CHUNK, CW), jnp.float32).at[:, 0].set(1.0)

    sums1, cnts = _sc_agg1(xstk, src2, dst1, zh, zc, onesc)

    b1r = b1.reshape(1, D)
    Ws2p = jnp.pad(W_self2, ((0, 0), (0, CP - NCLS)))
    Wn2p = jnp.pad(W_neigh2, ((0, 0), (0, CP - NCLS)))
    b2p = jnp.pad(b2, (0, CP - NCLS)).reshape(1, CP)
    p2, s2 = _tc_mid(x, sums1, cnts, W_self1, W_neigh1, b1r, Ws2p, Wn2p, b2p)

    sums2 = _sc_agg2(p2, srcw, dstw, zcp)
    return _tc_out(s2, sums2, cnts)
